# speedup vs baseline: 1.0212x; 1.0212x over previous
import functools

import jax
import jax.numpy as jnp
from jax import lax
from jax.experimental import pallas as pl
from jax.experimental.pallas import tpu as pltpu
from jax.experimental.pallas import tpu_sc as plsc

N = 10000
D = 128
BR = 400
NB = N // BR

_NC = 2
_NS = 16
_NW = _NC * _NS
_CH = 4
_CW = 80
_NPAD = _NW * _CH * _CW


def _sc_gather(idx3, table):
    mesh = plsc.VectorSubcoreMesh(core_axis_name="c", subcore_axis_name="s")

    @functools.partial(
        pl.kernel,
        mesh=mesh,
        out_type=jax.ShapeDtypeStruct((_NW, _CH, _CW, D), jnp.float32),
        scratch_types=[
            pltpu.VMEM((_CH, _CW), jnp.int32),
            pltpu.VMEM((_CH, _CW, D), jnp.float32),
            pltpu.SemaphoreType.DMA,
        ],
    )
    def gather_kernel(idx_hbm, table_hbm, out_hbm, idx_v, rows_v, sem):
        wid = lax.axis_index("s") * _NC + lax.axis_index("c")
        pltpu.sync_copy(idx_hbm.at[wid], idx_v)
        copies = [
            pltpu.async_copy(table_hbm.at[idx_v.at[j]], rows_v.at[j], sem)
            for j in range(_CH)
        ]
        for cp in copies:
            cp.wait()
        pltpu.sync_copy(rows_v, out_hbm.at[wid])

    return gather_kernel(idx3, table)


def _gcn_body(adj_ref, ue_ref, W1_ref, b1_ref, W2_ref, lw1_ref, lw2_ref,
              b2_ref, lb1_ref, lb2_ref, x_ref, v_s, wv_s):
    i = pl.program_id(0)

    @pl.when(i == 0)
    def _init():
        lw = jnp.dot(lw1_ref[...], lw2_ref[...],
                     preferred_element_type=jnp.float32)
        wv_s[...] = jnp.dot(W2_ref[...], lw,
                            preferred_element_type=jnp.float32)

    @pl.when(i < NB)
    def _phase1():
        au = jnp.dot(adj_ref[...], ue_ref[...],
                     preferred_element_type=jnp.float32)
        h = jnp.dot(au, W1_ref[...],
                    preferred_element_type=jnp.float32) + b1_ref[...]
        h = jnp.maximum(h, 0.0)
        v = jnp.dot(h, wv_s[...], preferred_element_type=jnp.float32)
        v_s[pl.ds(i * BR, BR), :] = v

    @pl.when(i >= NB)
    def _phase2():
        xv = jnp.dot(adj_ref[...], v_s[...],
                     preferred_element_type=jnp.float32)
        c = jnp.dot(
            jnp.dot(b2_ref[...], lw1_ref[...],
                    preferred_element_type=jnp.float32) + lb1_ref[...],
            lw2_ref[...], preferred_element_type=jnp.float32) + lb2_ref[...]
        x_ref[...] = xv + c


def _gcn_pallas(adj, ue, W1, b1, W2, lw1, lw2, b2, lb1, lb2):
    return pl.pallas_call(
        _gcn_body,
        grid=(2 * NB,),
        in_specs=[
            pl.BlockSpec((BR, N), lambda i: (lax.rem(i, NB), 0)),
            pl.BlockSpec((N, D), lambda i: (0, 0)),
            pl.BlockSpec((D, D), lambda i: (0, 0)),
            pl.BlockSpec((1, D), lambda i: (0, 0)),
            pl.BlockSpec((D, D), lambda i: (0, 0)),
            pl.BlockSpec((D, 16), lambda i: (0, 0)),
            pl.BlockSpec((16, 1), lambda i: (0, 0)),
            pl.BlockSpec((1, D), lambda i: (0, 0)),
            pl.BlockSpec((1, 16), lambda i: (0, 0)),
            pl.BlockSpec((1, 1), lambda i: (0, 0)),
        ],
        out_specs=pl.BlockSpec((BR, 1),
                               lambda i: (jnp.where(i < NB, 0, i - NB), 0)),
        out_shape=jax.ShapeDtypeStruct((N, 1), jnp.float32),
        scratch_shapes=[
            pltpu.VMEM((N, 1), jnp.float32),
            pltpu.VMEM((D, 1), jnp.float32),
        ],
        compiler_params=pltpu.CompilerParams(
            dimension_semantics=("arbitrary",),
        ),
    )(adj, ue, W1, b1, W2, lw1, lw2, b2, lb1, lb2)


def kernel(features, adj, emb_table, W1, b1, W2, b2, lw1, lb1, lw2, lb2):
    idx = features.astype(jnp.int32)
    idx3 = jnp.pad(idx, (0, _NPAD - N)).reshape(_NW, _CH, _CW)
    emb4 = _sc_gather(idx3, emb_table)
    user_emb = emb4.reshape(_NPAD, D)[:N]
    x = _gcn_pallas(adj, user_emb,
                    W1, b1.reshape(1, D), W2, lw1, lw2,
                    b2.reshape(1, D), lb1.reshape(1, 16), lb2.reshape(1, 1))
    return (x, user_emb)
